# Initial kernel scaffold; baseline (speedup 1.0000x reference)
import jax
import jax.numpy as jnp
from jax.experimental import pallas as pl


def kernel(idx, dictionary, feats):
    raise NotImplementedError("write your pallas kernel here")


# speedup vs baseline: 17.0397x; 17.0397x over previous
import functools

import jax
import jax.numpy as jnp
from jax import lax
from jax.experimental import pallas as pl
from jax.experimental.pallas import tpu as pltpu
from jax.experimental.pallas import tpu_sc as plsc

LEVELS = 16
FEATURE_DIM = 256
NUM_WORDS = 10000
DICT_SIZE = 256
BATCH = 8192


_W_BLK = 400


_CODE_W = 128


def _argmax_body(feats_ref, code_ref):
    cols = []
    for l in range(LEVELS):
        x = feats_ref[l]
        m = jnp.max(x, axis=-1, keepdims=True)
        lane_f = lax.broadcasted_iota(jnp.int32, x.shape, 1).astype(jnp.float32)
        cand = jnp.where(x == m, lane_f, float(DICT_SIZE))
        code_f = jnp.min(cand, axis=-1, keepdims=True)
        cols.append(code_f.astype(jnp.int32))
    cols.append(jnp.zeros((x.shape[0], _CODE_W - LEVELS), jnp.int32))
    code_ref[...] = jnp.concatenate(cols, axis=1)


def _compute_codes(feats):
    return pl.pallas_call(
        _argmax_body,
        grid=(NUM_WORDS // _W_BLK,),
        in_specs=[
            pl.BlockSpec((LEVELS, _W_BLK, DICT_SIZE), lambda i: (0, i, 0)),
        ],
        out_specs=pl.BlockSpec((_W_BLK, _CODE_W), lambda i: (i, 0)),
        out_shape=jax.ShapeDtypeStruct((NUM_WORDS, _CODE_W), jnp.int32),
    )(feats)


def _gather_codes(code_all, idx):
    info = plsc.get_sparse_core_info()
    nc, ns = info.num_cores, info.num_subcores
    nw = nc * ns
    b_per_w = BATCH // nw
    n_chunk = b_per_w // 128

    mesh = plsc.VectorSubcoreMesh(core_axis_name="c", subcore_axis_name="s")

    @functools.partial(
        pl.kernel,
        mesh=mesh,
        out_type=jax.ShapeDtypeStruct((BATCH, _CODE_W), jnp.int32),
        scratch_types=[
            pltpu.VMEM((n_chunk, 128), jnp.int32),
            pltpu.VMEM((b_per_w, _CODE_W), jnp.int32),
            pltpu.SemaphoreType.DMA,
        ],
    )
    def k(code_hbm, idx_hbm, out_hbm, idx_v, rows_v, sem):
        wid = lax.axis_index("s") * nc + lax.axis_index("c")
        base = wid * b_per_w
        for j in range(n_chunk):
            pltpu.sync_copy(idx_hbm.at[pl.ds(base + j * 128, 128)], idx_v.at[j])
        copies = [
            pltpu.async_copy(
                code_hbm.at[idx_v.at[j]], rows_v.at[pl.ds(j * 128, 128)], sem
            )
            for j in range(n_chunk)
        ]
        for c in copies:
            c.wait()
        pltpu.sync_copy(rows_v, out_hbm.at[pl.ds(base, b_per_w)])

    return k(code_all, idx)


_B_BLK = 512


def _expand_body(code_ref, dict_ref, out_ref):
    lane = lax.broadcasted_iota(jnp.int32, (_B_BLK, DICT_SIZE), 1)
    for l in range(LEVELS):
        c = code_ref[:, l : l + 1]
        oh = (c == lane).astype(jnp.float32)
        out_ref[:, l * FEATURE_DIM : (l + 1) * FEATURE_DIM] = jnp.dot(
            oh, dict_ref[l], preferred_element_type=jnp.float32
        )


def _expand(code_sel, dictionary):
    return pl.pallas_call(
        _expand_body,
        grid=(BATCH // _B_BLK,),
        in_specs=[
            pl.BlockSpec((_B_BLK, _CODE_W), lambda i: (i, 0)),
            pl.BlockSpec((LEVELS, DICT_SIZE, FEATURE_DIM), lambda i: (0, 0, 0)),
        ],
        out_specs=pl.BlockSpec((_B_BLK, LEVELS * FEATURE_DIM), lambda i: (i, 0)),
        out_shape=jax.ShapeDtypeStruct((BATCH, LEVELS * FEATURE_DIM), jnp.float32),
    )(code_sel, dictionary)


def kernel(idx, dictionary, feats):
    code_all = _compute_codes(feats)
    code_sel = _gather_codes(code_all, idx.astype(jnp.int32))
    return _expand(code_sel, dictionary)


# speedup vs baseline: 18.3682x; 1.0780x over previous
import functools

import jax
import jax.numpy as jnp
from jax import lax
from jax.experimental import pallas as pl
from jax.experimental.pallas import tpu as pltpu
from jax.experimental.pallas import tpu_sc as plsc

LEVELS = 16
FEATURE_DIM = 256
NUM_WORDS = 10000
DICT_SIZE = 256
BATCH = 8192


_W_BLK = 1000


_CODE_W = 128


def _argmax_body(feats_ref, code_ref):
    cols = []
    for l in range(LEVELS):
        x = feats_ref[l]
        m = jnp.max(x, axis=-1, keepdims=True)
        lane_f = lax.broadcasted_iota(jnp.int32, x.shape, 1).astype(jnp.float32)
        cand = jnp.where(x == m, lane_f, float(DICT_SIZE))
        code_f = jnp.min(cand, axis=-1, keepdims=True)
        cols.append(code_f.astype(jnp.int32))
    cols.append(jnp.zeros((x.shape[0], _CODE_W - LEVELS), jnp.int32))
    code_ref[...] = jnp.concatenate(cols, axis=1)


def _compute_codes(feats):
    return pl.pallas_call(
        _argmax_body,
        grid=(NUM_WORDS // _W_BLK,),
        in_specs=[
            pl.BlockSpec((LEVELS, _W_BLK, DICT_SIZE), lambda i: (0, i, 0)),
        ],
        out_specs=pl.BlockSpec((_W_BLK, _CODE_W), lambda i: (i, 0)),
        out_shape=jax.ShapeDtypeStruct((NUM_WORDS, _CODE_W), jnp.int32),
    )(feats)


def _gather_codes(code_all, idx):
    info = plsc.get_sparse_core_info()
    nc, ns = info.num_cores, info.num_subcores
    nw = nc * ns
    b_per_w = BATCH // nw
    n_chunk = b_per_w // 128

    mesh = plsc.VectorSubcoreMesh(core_axis_name="c", subcore_axis_name="s")

    @functools.partial(
        pl.kernel,
        mesh=mesh,
        out_type=jax.ShapeDtypeStruct((BATCH, _CODE_W), jnp.int32),
        scratch_types=[
            pltpu.VMEM((n_chunk, 128), jnp.int32),
            pltpu.VMEM((b_per_w, _CODE_W), jnp.int32),
            pltpu.SemaphoreType.DMA,
        ],
    )
    def k(code_hbm, idx_hbm, out_hbm, idx_v, rows_v, sem):
        wid = lax.axis_index("s") * nc + lax.axis_index("c")
        base = wid * b_per_w
        for j in range(n_chunk):
            pltpu.sync_copy(idx_hbm.at[pl.ds(base + j * 128, 128)], idx_v.at[j])
        copies = [
            pltpu.async_copy(
                code_hbm.at[idx_v.at[j]], rows_v.at[pl.ds(j * 128, 128)], sem
            )
            for j in range(n_chunk)
        ]
        for c in copies:
            c.wait()
        pltpu.sync_copy(rows_v, out_hbm.at[pl.ds(base, b_per_w)])

    return k(code_all, idx)


_B_BLK = 1024


def _expand_body(code_ref, dict_ref, out_ref):
    lane = lax.broadcasted_iota(jnp.int32, (_B_BLK, DICT_SIZE), 1)
    for l in range(LEVELS):
        c = code_ref[:, l : l + 1]
        oh = (c == lane).astype(jnp.float32)
        out_ref[:, l * FEATURE_DIM : (l + 1) * FEATURE_DIM] = jnp.dot(
            oh, dict_ref[l], preferred_element_type=jnp.float32
        )


def _expand(code_sel, dictionary):
    return pl.pallas_call(
        _expand_body,
        grid=(BATCH // _B_BLK,),
        in_specs=[
            pl.BlockSpec((_B_BLK, _CODE_W), lambda i: (i, 0)),
            pl.BlockSpec((LEVELS, DICT_SIZE, FEATURE_DIM), lambda i: (0, 0, 0)),
        ],
        out_specs=pl.BlockSpec((_B_BLK, LEVELS * FEATURE_DIM), lambda i: (i, 0)),
        out_shape=jax.ShapeDtypeStruct((BATCH, LEVELS * FEATURE_DIM), jnp.float32),
    )(code_sel, dictionary)


def kernel(idx, dictionary, feats):
    code_all = _compute_codes(feats)
    code_sel = _gather_codes(code_all, idx.astype(jnp.int32))
    return _expand(code_sel, dictionary)


# speedup vs baseline: 18.5563x; 1.0102x over previous
import functools

import jax
import jax.numpy as jnp
from jax import lax
from jax.experimental import pallas as pl
from jax.experimental.pallas import tpu as pltpu
from jax.experimental.pallas import tpu_sc as plsc

LEVELS = 16
FEATURE_DIM = 256
NUM_WORDS = 10000
DICT_SIZE = 256
BATCH = 8192


_W_BLK = 1000


_CODE_W = 128


def _argmax_body(feats_ref, code_ref):
    cols = []
    for l in range(LEVELS):
        x = feats_ref[l]
        m = jnp.max(x, axis=-1, keepdims=True)
        lane_f = lax.broadcasted_iota(jnp.int32, x.shape, 1).astype(jnp.float32)
        cand = jnp.where(x == m, lane_f, float(DICT_SIZE))
        code_f = jnp.min(cand, axis=-1, keepdims=True)
        cols.append(code_f.astype(jnp.int32))
    cols.append(jnp.zeros((x.shape[0], _CODE_W - LEVELS), jnp.int32))
    code_ref[...] = jnp.concatenate(cols, axis=1)


def _compute_codes(feats):
    return pl.pallas_call(
        _argmax_body,
        grid=(NUM_WORDS // _W_BLK,),
        in_specs=[
            pl.BlockSpec((LEVELS, _W_BLK, DICT_SIZE), lambda i: (0, i, 0)),
        ],
        out_specs=pl.BlockSpec((_W_BLK, _CODE_W), lambda i: (i, 0)),
        out_shape=jax.ShapeDtypeStruct((NUM_WORDS, _CODE_W), jnp.int32),
    )(feats)


def _gather_codes(code_all, idx):
    info = plsc.get_sparse_core_info()
    nc, ns = info.num_cores, info.num_subcores
    nw = nc * ns
    b_per_w = BATCH // nw
    n_chunk = b_per_w // 128

    mesh = plsc.VectorSubcoreMesh(core_axis_name="c", subcore_axis_name="s")

    @functools.partial(
        pl.kernel,
        mesh=mesh,
        out_type=jax.ShapeDtypeStruct((BATCH, _CODE_W), jnp.int32),
        scratch_types=[
            pltpu.VMEM((n_chunk, 128), jnp.int32),
            pltpu.VMEM((b_per_w, _CODE_W), jnp.int32),
            pltpu.SemaphoreType.DMA,
        ],
    )
    def k(code_hbm, idx_hbm, out_hbm, idx_v, rows_v, sem):
        wid = lax.axis_index("s") * nc + lax.axis_index("c")
        base = wid * b_per_w
        for j in range(n_chunk):
            pltpu.sync_copy(idx_hbm.at[pl.ds(base + j * 128, 128)], idx_v.at[j])
        copies = [
            pltpu.async_copy(
                code_hbm.at[idx_v.at[j]], rows_v.at[pl.ds(j * 128, 128)], sem
            )
            for j in range(n_chunk)
        ]
        for c in copies:
            c.wait()
        pltpu.sync_copy(rows_v, out_hbm.at[pl.ds(base, b_per_w)])

    return k(code_all, idx)


_B_BLK = 512


def _expand_body(code_ref, dict_ref, out_ref):
    lane = lax.broadcasted_iota(jnp.int32, (_B_BLK, DICT_SIZE), 1)
    for l in range(LEVELS):
        c = code_ref[:, l : l + 1]
        oh = (c == lane).astype(jnp.float32)
        out_ref[:, l * FEATURE_DIM : (l + 1) * FEATURE_DIM] = jnp.dot(
            oh, dict_ref[l], preferred_element_type=jnp.float32
        )


def _expand(code_sel, dictionary):
    return pl.pallas_call(
        _expand_body,
        grid=(BATCH // _B_BLK,),
        in_specs=[
            pl.BlockSpec((_B_BLK, _CODE_W), lambda i: (i, 0)),
            pl.BlockSpec((LEVELS, DICT_SIZE, FEATURE_DIM), lambda i: (0, 0, 0)),
        ],
        out_specs=pl.BlockSpec((_B_BLK, LEVELS * FEATURE_DIM), lambda i: (i, 0)),
        out_shape=jax.ShapeDtypeStruct((BATCH, LEVELS * FEATURE_DIM), jnp.float32),
    )(code_sel, dictionary)


def kernel(idx, dictionary, feats):
    code_all = _compute_codes(feats)
    code_sel = _gather_codes(code_all, idx.astype(jnp.int32))
    return _expand(code_sel, dictionary)
